# Initial kernel scaffold; baseline (speedup 1.0000x reference)
import jax
import jax.numpy as jnp
from jax.experimental import pallas as pl


def kernel(x, weights):
    raise NotImplementedError("write your pallas kernel here")


# speedup vs baseline: 1.0648x; 1.0648x over previous
import functools

import jax
import jax.numpy as jnp
from jax import lax
from jax.experimental import pallas as pl
from jax.experimental.pallas import tpu as pltpu
from jax.experimental.pallas import tpu_sc as plsc

_GRID_W = 50
_K = 2500
_KPAD = 2560
_D = 128
_B = 8192
_BB = 512
_G = _B // _BB
_KSPLIT = 1256


def _bmu_body(x_ref, x2_ref, w_ref, w2_ref, bmu_ref, li_ref, lj_ref):
    x = x_ref[...]
    w = w_ref[...]
    dot = lax.dot_general(
        x.astype(jnp.bfloat16), w.astype(jnp.bfloat16),
        (((1,), (1,)), ((), ())),
        preferred_element_type=jnp.float32,
    )
    x2 = x2_ref[0]
    w2 = w2_ref[0]
    d = x2 - 2.0 * dot
    d = d + w2
    kid = lax.broadcasted_iota(jnp.int32, d.shape, 1)
    inf = jnp.float32(jnp.inf)
    m1 = kid < _KSPLIT
    d1 = jnp.where(m1, d, inf)
    d2c = jnp.where(m1, inf, d)
    v1 = jnp.min(d1, axis=1, keepdims=True)
    i1 = jnp.min(jnp.where(d1 == v1, kid, _KPAD), axis=1)
    v2 = jnp.min(d2c, axis=1, keepdims=True)
    c1 = v1.astype(jnp.bfloat16).astype(jnp.float32)
    take2 = (v2 < c1) & (d2c == v2)
    i2sel = jnp.min(jnp.where(take2, kid, _KPAD), axis=1)
    bmu = jnp.where(i2sel < _KPAD, i2sel, i1)
    bmu_ref[0, 0, :] = bmu
    li_ref[0, 0, :] = lax.div(bmu, _GRID_W).astype(jnp.float32)
    lj_ref[0, 0, :] = lax.rem(bmu, _GRID_W).astype(jnp.float32)


def _bmu_call(x, x2_3d, wpad, w2_3d):
    return pl.pallas_call(
        _bmu_body,
        grid=(_G,),
        in_specs=[
            pl.BlockSpec((_BB, _D), lambda g: (g, 0)),
            pl.BlockSpec((1, _BB, 1), lambda g: (g, 0, 0)),
            pl.BlockSpec((_KPAD, _D), lambda g: (0, 0)),
            pl.BlockSpec((1, 1, _KPAD), lambda g: (0, 0, 0)),
        ],
        out_specs=[
            pl.BlockSpec((1, 1, _BB), lambda g: (g, 0, 0)),
            pl.BlockSpec((1, 1, _BB), lambda g: (g, 0, 0)),
            pl.BlockSpec((1, 1, _BB), lambda g: (g, 0, 0)),
        ],
        out_shape=[
            jax.ShapeDtypeStruct((_G, 1, _BB), jnp.int32),
            jax.ShapeDtypeStruct((_G, 1, _BB), jnp.float32),
            jax.ShapeDtypeStruct((_G, 1, _BB), jnp.float32),
        ],
    )(x, x2_3d, wpad, w2_3d)


@functools.lru_cache(maxsize=None)
def _make_sc_gather():
    info = plsc.get_sparse_core_info()
    nc, ns = info.num_cores, info.num_subcores
    nw = nc * ns
    bpw = _B // nw
    ch = 128
    nch = bpw // ch
    mesh = plsc.VectorSubcoreMesh(core_axis_name="c", subcore_axis_name="s")

    @functools.partial(
        pl.kernel,
        mesh=mesh,
        out_type=jax.ShapeDtypeStruct((_B, _D), jnp.float32),
        scratch_types=[
            pltpu.VMEM((nch, ch), jnp.int32),
            pltpu.VMEM((ch, _D), jnp.float32),
            pltpu.SemaphoreType.DMA,
        ],
    )
    def gather(table_hbm, idx_hbm, out_hbm, idx_v, rows_v, sem):
        wid = lax.axis_index("s") * nc + lax.axis_index("c")
        base = wid * bpw
        for c in range(nch):
            off = base + c * ch
            pltpu.sync_copy(idx_hbm.at[pl.ds(off, ch)], idx_v.at[c])
            pltpu.async_copy(table_hbm.at[idx_v.at[c]], rows_v, sem).wait()
            pltpu.sync_copy(rows_v, out_hbm.at[pl.ds(off, ch)])

    return gather


def kernel(x, weights):
    x2 = jnp.sum(x * x, axis=1, keepdims=True)
    w2 = jnp.sum(weights * weights, axis=1)
    wpad = jnp.pad(weights, ((0, _KPAD - _K), (0, 0)))
    w2_3d = jnp.pad(w2, (0, _KPAD - _K), constant_values=jnp.inf).reshape(
        1, 1, _KPAD
    )
    x2_3d = x2.reshape(_G, _BB, 1)
    bmu3, li3, lj3 = _bmu_call(x, x2_3d, wpad, w2_3d)
    bmu = bmu3.reshape(_B)
    quantized = _make_sc_gather()(weights, bmu)
    bmu_locs = jnp.stack([li3.reshape(_B), lj3.reshape(_B)], axis=1)
    return quantized, bmu_locs


# speedup vs baseline: 1.1151x; 1.0473x over previous
import functools

import jax
import jax.numpy as jnp
from jax import lax
from jax.experimental import pallas as pl
from jax.experimental.pallas import tpu as pltpu
from jax.experimental.pallas import tpu_sc as plsc

_GRID_W = 50
_K = 2500
_KPAD = 2560
_D = 128
_B = 8192
_BB = 512
_G = _B // _BB
_KSPLIT = 1256


def _bmu_body(x_ref, x2_ref, w_ref, w2a_ref, w2b_ref, kidf_ref, bmu_ref, li_ref, lj_ref):
    x = x_ref[...]
    w = w_ref[...]
    xs = x + x
    dot2 = lax.dot_general(
        xs.astype(jnp.bfloat16), w.astype(jnp.bfloat16),
        (((1,), (1,)), ((), ())),
        preferred_element_type=jnp.float32,
    )
    x2 = x2_ref[0]
    w2a = w2a_ref[0]
    w2b = w2b_ref[0]
    base = x2 - dot2
    v1 = jnp.min(base + w2a, axis=1, keepdims=True)
    v2 = jnp.min(base + w2b, axis=1, keepdims=True)
    c1 = v1.astype(jnp.bfloat16).astype(jnp.float32)
    cond = v2 < c1
    vw = jnp.where(cond, v2, v1)
    w2sel = jnp.where(cond, w2b, w2a)
    kidf = kidf_ref[0]
    bmuf = jnp.min(
        jnp.where(base + w2sel == vw, kidf, jnp.float32(_KPAD)), axis=1
    )
    bmu = bmuf.astype(jnp.int32)
    bmu_ref[0, 0, :] = bmu
    li_ref[0, 0, :] = lax.div(bmu, _GRID_W).astype(jnp.float32)
    lj_ref[0, 0, :] = lax.rem(bmu, _GRID_W).astype(jnp.float32)


def _bmu_call(x, x2_3d, wpad, w2a_3d, w2b_3d, kidf_3d):
    return pl.pallas_call(
        _bmu_body,
        grid=(_G,),
        in_specs=[
            pl.BlockSpec((_BB, _D), lambda g: (g, 0)),
            pl.BlockSpec((1, _BB, 1), lambda g: (g, 0, 0)),
            pl.BlockSpec((_KPAD, _D), lambda g: (0, 0)),
            pl.BlockSpec((1, 1, _KPAD), lambda g: (0, 0, 0)),
            pl.BlockSpec((1, 1, _KPAD), lambda g: (0, 0, 0)),
            pl.BlockSpec((1, 1, _KPAD), lambda g: (0, 0, 0)),
        ],
        out_specs=[
            pl.BlockSpec((1, 1, _BB), lambda g: (g, 0, 0)),
            pl.BlockSpec((1, 1, _BB), lambda g: (g, 0, 0)),
            pl.BlockSpec((1, 1, _BB), lambda g: (g, 0, 0)),
        ],
        out_shape=[
            jax.ShapeDtypeStruct((_G, 1, _BB), jnp.int32),
            jax.ShapeDtypeStruct((_G, 1, _BB), jnp.float32),
            jax.ShapeDtypeStruct((_G, 1, _BB), jnp.float32),
        ],
    )(x, x2_3d, wpad, w2a_3d, w2b_3d, kidf_3d)


@functools.lru_cache(maxsize=None)
def _make_sc_gather():
    info = plsc.get_sparse_core_info()
    nc, ns = info.num_cores, info.num_subcores
    nw = nc * ns
    bpw = _B // nw
    ch = 128
    nch = bpw // ch
    mesh = plsc.VectorSubcoreMesh(core_axis_name="c", subcore_axis_name="s")

    @functools.partial(
        pl.kernel,
        mesh=mesh,
        out_type=jax.ShapeDtypeStruct((_B, _D), jnp.float32),
        scratch_types=[
            pltpu.VMEM((nch, ch), jnp.int32),
            pltpu.VMEM((ch, _D), jnp.float32),
            pltpu.SemaphoreType.DMA,
        ],
    )
    def gather(table_hbm, idx_hbm, out_hbm, idx_v, rows_v, sem):
        wid = lax.axis_index("s") * nc + lax.axis_index("c")
        base = wid * bpw
        for c in range(nch):
            off = base + c * ch
            pltpu.sync_copy(idx_hbm.at[pl.ds(off, ch)], idx_v.at[c])
            pltpu.async_copy(table_hbm.at[idx_v.at[c]], rows_v, sem).wait()
            pltpu.sync_copy(rows_v, out_hbm.at[pl.ds(off, ch)])

    return gather


def kernel(x, weights):
    x2 = jnp.sum(x * x, axis=1, keepdims=True)
    w2 = jnp.sum(weights * weights, axis=1)
    wpad = jnp.pad(weights, ((0, _KPAD - _K), (0, 0)))
    w2p = jnp.pad(w2, (0, _KPAD - _K), constant_values=jnp.inf)
    karange = jnp.arange(_KPAD)
    inf = jnp.float32(jnp.inf)
    w2a_3d = jnp.where(karange < _KSPLIT, w2p, inf).reshape(1, 1, _KPAD)
    w2b_3d = jnp.where(karange < _KSPLIT, inf, w2p).reshape(1, 1, _KPAD)
    kidf_3d = karange.astype(jnp.float32).reshape(1, 1, _KPAD)
    x2_3d = x2.reshape(_G, _BB, 1)
    bmu3, li3, lj3 = _bmu_call(x, x2_3d, wpad, w2a_3d, w2b_3d, kidf_3d)
    bmu = bmu3.reshape(_B)
    quantized = _make_sc_gather()(weights, bmu)
    bmu_locs = jnp.stack([li3.reshape(_B), lj3.reshape(_B)], axis=1)
    return quantized, bmu_locs


# speedup vs baseline: 1.1264x; 1.0101x over previous
import functools

import jax
import jax.numpy as jnp
from jax import lax
from jax.experimental import pallas as pl
from jax.experimental.pallas import tpu as pltpu
from jax.experimental.pallas import tpu_sc as plsc

_GRID_W = 50
_K = 2500
_KPAD = 2560
_D = 128
_B = 8192
_BB = 1024
_G = _B // _BB
_KSPLIT = 1256


def _bmu_body(x_ref, x2_ref, w_ref, w2a_ref, w2b_ref, kidf_ref, bmu_ref, li_ref, lj_ref):
    x = x_ref[...]
    w = w_ref[...]
    xs = x + x
    dot2 = lax.dot_general(
        xs.astype(jnp.bfloat16), w.astype(jnp.bfloat16),
        (((1,), (1,)), ((), ())),
        preferred_element_type=jnp.float32,
    )
    x2 = x2_ref[0]
    w2a = w2a_ref[0]
    w2b = w2b_ref[0]
    base = x2 - dot2
    v1 = jnp.min(base + w2a, axis=1, keepdims=True)
    v2 = jnp.min(base + w2b, axis=1, keepdims=True)
    c1 = v1.astype(jnp.bfloat16).astype(jnp.float32)
    cond = v2 < c1
    vw = jnp.where(cond, v2, v1)
    w2sel = jnp.where(cond, w2b, w2a)
    kidf = kidf_ref[0]
    bmuf = jnp.min(
        jnp.where(base + w2sel == vw, kidf, jnp.float32(_KPAD)), axis=1
    )
    bmu = bmuf.astype(jnp.int32)
    bmu_ref[0, 0, :] = bmu
    li_ref[0, 0, :] = lax.div(bmu, _GRID_W).astype(jnp.float32)
    lj_ref[0, 0, :] = lax.rem(bmu, _GRID_W).astype(jnp.float32)


def _bmu_call(x, x2_3d, wpad, w2a_3d, w2b_3d, kidf_3d):
    return pl.pallas_call(
        _bmu_body,
        grid=(_G,),
        in_specs=[
            pl.BlockSpec((_BB, _D), lambda g: (g, 0)),
            pl.BlockSpec((1, _BB, 1), lambda g: (g, 0, 0)),
            pl.BlockSpec((_KPAD, _D), lambda g: (0, 0)),
            pl.BlockSpec((1, 1, _KPAD), lambda g: (0, 0, 0)),
            pl.BlockSpec((1, 1, _KPAD), lambda g: (0, 0, 0)),
            pl.BlockSpec((1, 1, _KPAD), lambda g: (0, 0, 0)),
        ],
        out_specs=[
            pl.BlockSpec((1, 1, _BB), lambda g: (g, 0, 0)),
            pl.BlockSpec((1, 1, _BB), lambda g: (g, 0, 0)),
            pl.BlockSpec((1, 1, _BB), lambda g: (g, 0, 0)),
        ],
        out_shape=[
            jax.ShapeDtypeStruct((_G, 1, _BB), jnp.int32),
            jax.ShapeDtypeStruct((_G, 1, _BB), jnp.float32),
            jax.ShapeDtypeStruct((_G, 1, _BB), jnp.float32),
        ],
    )(x, x2_3d, wpad, w2a_3d, w2b_3d, kidf_3d)


@functools.lru_cache(maxsize=None)
def _make_sc_gather():
    info = plsc.get_sparse_core_info()
    nc, ns = info.num_cores, info.num_subcores
    nw = nc * ns
    bpw = _B // nw
    ch = 128
    nch = bpw // ch
    mesh = plsc.VectorSubcoreMesh(core_axis_name="c", subcore_axis_name="s")

    @functools.partial(
        pl.kernel,
        mesh=mesh,
        out_type=jax.ShapeDtypeStruct((_B, _D), jnp.float32),
        scratch_types=[
            pltpu.VMEM((nch, ch), jnp.int32),
            pltpu.VMEM((ch, _D), jnp.float32),
            pltpu.SemaphoreType.DMA,
        ],
    )
    def gather(table_hbm, idx_hbm, out_hbm, idx_v, rows_v, sem):
        wid = lax.axis_index("s") * nc + lax.axis_index("c")
        base = wid * bpw
        for c in range(nch):
            off = base + c * ch
            pltpu.sync_copy(idx_hbm.at[pl.ds(off, ch)], idx_v.at[c])
            pltpu.async_copy(table_hbm.at[idx_v.at[c]], rows_v, sem).wait()
            pltpu.sync_copy(rows_v, out_hbm.at[pl.ds(off, ch)])

    return gather


def kernel(x, weights):
    x2 = jnp.sum(x * x, axis=1, keepdims=True)
    w2 = jnp.sum(weights * weights, axis=1)
    wpad = jnp.pad(weights, ((0, _KPAD - _K), (0, 0)))
    w2p = jnp.pad(w2, (0, _KPAD - _K), constant_values=jnp.inf)
    karange = jnp.arange(_KPAD)
    inf = jnp.float32(jnp.inf)
    w2a_3d = jnp.where(karange < _KSPLIT, w2p, inf).reshape(1, 1, _KPAD)
    w2b_3d = jnp.where(karange < _KSPLIT, inf, w2p).reshape(1, 1, _KPAD)
    kidf_3d = karange.astype(jnp.float32).reshape(1, 1, _KPAD)
    x2_3d = x2.reshape(_G, _BB, 1)
    bmu3, li3, lj3 = _bmu_call(x, x2_3d, wpad, w2a_3d, w2b_3d, kidf_3d)
    bmu = bmu3.reshape(_B)
    quantized = _make_sc_gather()(weights, bmu)
    bmu_locs = jnp.stack([li3.reshape(_B), lj3.reshape(_B)], axis=1)
    return quantized, bmu_locs
